# Initial kernel scaffold; baseline (speedup 1.0000x reference)
import jax
import jax.numpy as jnp
from jax.experimental import pallas as pl


def kernel(X, Omega, W, H, lambda_a, i):
    raise NotImplementedError("write your pallas kernel here")


# speedup vs baseline: 32.2519x; 32.2519x over previous
import jax
import jax.numpy as jnp
from jax.experimental import pallas as pl
from jax.experimental.pallas import tpu as pltpu

N_ROWS = 4096
N_COLS = 1024
CHUNK = 512
N_CHUNKS = N_ROWS // CHUNK
N_TOTAL = N_ROWS * N_COLS
NEG_LN_EPS = 2.3025850929940455
BISECT_ITERS = 22


def _body(x_ref, om_ref, w_ref, h_ref, lam_ref, a_ref, lamout_ref):
    f32 = jnp.float32

    def p1(c, carry):
        s1, s2, mx, cm = carry
        sl = pl.ds(c * CHUNK, CHUNK)
        om = om_ref[sl, :].astype(f32)
        wh = jnp.dot(w_ref[sl, :], h_ref[:, :], preferred_element_type=f32)
        a = (x_ref[sl, :] - wh) * om
        a_ref[sl, :] = a
        ab = jnp.abs(a)
        return (s1 + jnp.sum(ab), s2 + jnp.sum(ab * ab),
                jnp.maximum(mx, jnp.max(ab)), cm + jnp.sum(om))

    zero = f32(0.0)
    s1, s2, mx, cnt = jax.lax.fori_loop(0, N_CHUNKS, p1,
                                        (zero, zero, zero, zero))

    mean = s1 / cnt
    varsum = s2 - 2.0 * mean * s1 + cnt * mean * mean
    n_std = jnp.sqrt(varsum / (cnt - 1.0))

    miss = f32(N_TOTAL) - cnt
    t25 = 0.25 * (cnt - 1.0)
    t75 = 0.75 * (cnt - 1.0)

    def bis(_, carry):
        lo25, hi25, lo75, hi75 = carry
        m25 = 0.5 * (lo25 + hi25)
        m75 = 0.5 * (lo75 + hi75)

        def cp(c, cc):
            c25, c75 = cc
            sl = pl.ds(c * CHUNK, CHUNK)
            ab = jnp.abs(a_ref[sl, :])
            return (c25 + jnp.sum(jnp.where(ab <= m25, 1.0, 0.0)),
                    c75 + jnp.sum(jnp.where(ab <= m75, 1.0, 0.0)))

        c25, c75 = jax.lax.fori_loop(0, N_CHUNKS, cp, (zero, zero))
        c25 = c25 - miss
        c75 = c75 - miss
        gt25 = c25 > t25
        gt75 = c75 > t75
        return (jnp.where(gt25, lo25, m25), jnp.where(gt25, m25, hi25),
                jnp.where(gt75, lo75, m75), jnp.where(gt75, m75, hi75))

    lo25, hi25, lo75, hi75 = jax.lax.fori_loop(
        0, BISECT_ITERS, bis, (zero, mx, zero, mx))
    q25 = 0.5 * (lo25 + hi25)
    q75 = 0.5 * (lo75 + hi75)
    iqr = q75 - q25

    deta2 = (1.06 * jnp.minimum(n_std, iqr / 1.34)
             * jnp.exp(-0.2 * jnp.log(cnt)))
    thr = deta2 * NEG_LN_EPS

    def lp(c, lam):
        sl = pl.ds(c * CHUNK, CHUNK)
        ab = jnp.abs(a_ref[sl, :])
        cand = jnp.min(jnp.where(ab > thr, ab * ab, jnp.inf))
        return jnp.minimum(lam, cand)

    lam_cand = jax.lax.fori_loop(0, N_CHUNKS, lp, f32(jnp.inf))
    lambda_new = jnp.minimum(lam_cand, lam_ref[0])
    tcut = jnp.sqrt(lambda_new)

    def op(c, _):
        sl = pl.ds(c * CHUNK, CHUNK)
        a = a_ref[sl, :]
        a_ref[sl, :] = jnp.where(jnp.abs(a) < tcut, 0.0, a)
        return 0

    jax.lax.fori_loop(0, N_CHUNKS, op, 0)
    lamout_ref[0] = lambda_new


def _call(x_i, om8, w, h, lam, interpret=False):
    return pl.pallas_call(
        _body,
        out_shape=[
            jax.ShapeDtypeStruct((N_ROWS, N_COLS), jnp.float32),
            jax.ShapeDtypeStruct((1,), jnp.float32),
        ],
        in_specs=[
            pl.BlockSpec(memory_space=pltpu.VMEM),
            pl.BlockSpec(memory_space=pltpu.VMEM),
            pl.BlockSpec(memory_space=pltpu.VMEM),
            pl.BlockSpec(memory_space=pltpu.VMEM),
            pl.BlockSpec(memory_space=pltpu.SMEM),
        ],
        out_specs=[
            pl.BlockSpec(memory_space=pltpu.VMEM),
            pl.BlockSpec(memory_space=pltpu.SMEM),
        ],
        interpret=interpret,
    )(x_i, om8, w, h, lam)


def kernel(X, Omega, W, H, lambda_a, i):
    x_i = X[i]
    om8 = Omega.astype(jnp.int8)
    lam = jnp.reshape(lambda_a.astype(jnp.float32), (1,))
    a, lam_new = _call(x_i, om8, W, H, lam)
    return (a, lam_new[0])
